# Initial kernel scaffold; baseline (speedup 1.0000x reference)
import jax
import jax.numpy as jnp
from jax.experimental import pallas as pl


def kernel(x, table):
    raise NotImplementedError("write your pallas kernel here")


# speedup vs baseline: 2.9537x; 2.9537x over previous
import functools

import jax
import jax.numpy as jnp
from jax import lax
from jax.experimental import pallas as pl
from jax.experimental.pallas import tpu as pltpu
from jax.experimental.pallas import tpu_sc as plsc

_B = 1024
_F = 330
_L = 20
_E = 16
_SEG = _B * _F
_NW = 32
_SEG_W = _SEG // _NW
_S = 96
_IDS_CHUNK = _S * _L
_DMA_ROWS = 128
_NDMA = _IDS_CHUNK // _DMA_ROWS
_NCHUNK = _SEG_W // _S
_IDXROW_W = _SEG_W * _L // _DMA_ROWS


def _emb_body(x_hbm, table_hbm, out_hbm, idx_v, rows_v, acc_v, sem):
    nc = 2
    wid = lax.axis_index("s") * nc + lax.axis_index("c")
    i0 = wid * _SEG_W * _L
    o0 = wid * _SEG_W * _E

    def chunk_body(g, carry):
        pltpu.sync_copy(x_hbm.at[pl.ds(i0 + g * _IDS_CHUNK, _IDS_CHUNK)], idx_v)
        for j in range(_NDMA):
            pltpu.async_copy(
                table_hbm.at[idx_v.at[pl.ds(j * _DMA_ROWS, _DMA_ROWS)]],
                rows_v.at[pl.ds(j * _DMA_ROWS, _DMA_ROWS)],
                sem,
            )
        for j in range(_NDMA):
            pltpu.make_async_copy(
                table_hbm.at[idx_v.at[pl.ds(j * _DMA_ROWS, _DMA_ROWS)]],
                rows_v.at[pl.ds(j * _DMA_ROWS, _DMA_ROWS)],
                sem,
            ).wait()

        def seg_body(i, c):
            base = i * _L
            acc = rows_v[base, :]
            for l in range(1, _L):
                acc = acc + rows_v[base + l, :]
            acc_v[pl.ds(i * _E, _E)] = acc
            return c

        lax.fori_loop(0, _S, seg_body, 0)

        pltpu.sync_copy(acc_v, out_hbm.at[pl.ds(o0 + g * _S * _E, _S * _E)])
        return carry

    lax.fori_loop(0, _NCHUNK, chunk_body, 0)


@jax.jit
def _emb_lookup(x2, table):
    mesh = plsc.VectorSubcoreMesh(core_axis_name="c", subcore_axis_name="s")
    f = pl.kernel(
        _emb_body,
        out_type=jax.ShapeDtypeStruct((_SEG * _E,), jnp.float32),
        mesh=mesh,
        scratch_types=[
            pltpu.VMEM((_IDS_CHUNK,), jnp.int32),
            pltpu.VMEM((_IDS_CHUNK, _E), jnp.float32),
            pltpu.VMEM((_S * _E,), jnp.float32),
            pltpu.SemaphoreType.DMA,
        ],
        compiler_params=pltpu.CompilerParams(use_tc_tiling_on_sc=False),
    )
    return f(x2, table)


def kernel(x, table):
    out = _emb_lookup(x.reshape(-1), table)
    return out.reshape(_B, _F * _E)


# speedup vs baseline: 3.1255x; 1.0582x over previous
import functools

import jax
import jax.numpy as jnp
from jax import lax
from jax.experimental import pallas as pl
from jax.experimental.pallas import tpu as pltpu
from jax.experimental.pallas import tpu_sc as plsc

_B = 1024
_F = 330
_L = 20
_E = 16
_SEG = _B * _F
_NW = 32
_SEG_W = _SEG // _NW
_S = 96
_IDS_CHUNK = _S * _L
_DMA_ROWS = 128
_NDMA = _IDS_CHUNK // _DMA_ROWS
_NCHUNK = _SEG_W // _S
_NPAIR = _NCHUNK // 2


def _emb_body(x_hbm, table_hbm, out_hbm,
              idx0, idx1, rows0, rows1, acc_v, sem0, sem1):
    nc = 2
    wid = lax.axis_index("s") * nc + lax.axis_index("c")
    i0 = wid * _SEG_W * _L
    o0 = wid * _SEG_W * _E

    def load_idx(g, idx_v):
        pltpu.sync_copy(x_hbm.at[pl.ds(i0 + g * _IDS_CHUNK, _IDS_CHUNK)], idx_v)

    def fire(idx_v, rows_v, sem):
        for j in range(_NDMA):
            pltpu.async_copy(
                table_hbm.at[idx_v.at[pl.ds(j * _DMA_ROWS, _DMA_ROWS)]],
                rows_v.at[pl.ds(j * _DMA_ROWS, _DMA_ROWS)],
                sem,
            )

    def drain(idx_v, rows_v, sem):
        for j in range(_NDMA):
            pltpu.make_async_copy(
                table_hbm.at[idx_v.at[pl.ds(j * _DMA_ROWS, _DMA_ROWS)]],
                rows_v.at[pl.ds(j * _DMA_ROWS, _DMA_ROWS)],
                sem,
            ).wait()

    def pool(g, rows_v):
        def seg_body(i, c):
            base = i * _L
            acc = rows_v[base, :]
            for l in range(1, _L):
                acc = acc + rows_v[base + l, :]
            acc_v[pl.ds(i * _E, _E)] = acc
            return c

        lax.fori_loop(0, _S, seg_body, 0)
        pltpu.sync_copy(acc_v, out_hbm.at[pl.ds(o0 + g * _S * _E, _S * _E)])

    load_idx(0, idx0)
    fire(idx0, rows0, sem0)

    def pair_body(t, carry):
        a = 2 * t
        load_idx(a + 1, idx1)
        fire(idx1, rows1, sem1)
        drain(idx0, rows0, sem0)
        pool(a, rows0)

        @pl.when(t < _NPAIR - 1)
        def _():
            load_idx(a + 2, idx0)
            fire(idx0, rows0, sem0)

        drain(idx1, rows1, sem1)
        pool(a + 1, rows1)
        return carry

    lax.fori_loop(0, _NPAIR, pair_body, 0)


@jax.jit
def _emb_lookup(x2, table):
    mesh = plsc.VectorSubcoreMesh(core_axis_name="c", subcore_axis_name="s")
    f = pl.kernel(
        _emb_body,
        out_type=jax.ShapeDtypeStruct((_SEG * _E,), jnp.float32),
        mesh=mesh,
        scratch_types=[
            pltpu.VMEM((_IDS_CHUNK,), jnp.int32),
            pltpu.VMEM((_IDS_CHUNK,), jnp.int32),
            pltpu.VMEM((_IDS_CHUNK, _E), jnp.float32),
            pltpu.VMEM((_IDS_CHUNK, _E), jnp.float32),
            pltpu.VMEM((_S * _E,), jnp.float32),
            pltpu.SemaphoreType.DMA,
            pltpu.SemaphoreType.DMA,
        ],
        compiler_params=pltpu.CompilerParams(use_tc_tiling_on_sc=False),
    )
    return f(x2, table)


def kernel(x, table):
    out = _emb_lookup(x.reshape(-1), table)
    return out.reshape(_B, _F * _E)
